# baseline (speedup 1.0000x reference)
import jax
import jax.numpy as jnp
from jax import lax
from jax.experimental import pallas as pl
from jax.experimental.pallas import tpu as pltpu
from jax.experimental.pallas import tpu_sc as plsc

NUM_MODELS = 2
NUM_BOXES = 100000
DIM = 64
BATCH = 16384

_L = 16
_NC, _NS = 2, 16
_NW = _NC * _NS
_BPW = BATCH // _NW
_CHUNK = 128
_NCH = _BPW // _CHUNK


def _sc_body(zf, ldf, ids3, out, ids_v, ids1_v, z_v, ld_v, sem):
    wid = lax.axis_index("s") * _NC + lax.axis_index("c")
    base = wid * _BPW
    pltpu.sync_copy(ids3.at[wid], ids_v)
    for j in range(_NCH):
        for i in range(_CHUNK // _L):
            s = pl.ds(i * _L, _L)
            ids1_v[j, s] = ids_v[j, s] + NUM_BOXES
    for m, idx in ((0, ids_v), (1, ids1_v)):
        copies = []
        for j in range(_NCH):
            dst = pl.ds(j * _CHUNK, _CHUNK)
            copies.append(pltpu.async_copy(zf.at[idx.at[j]], z_v.at[dst], sem))
            copies.append(pltpu.async_copy(ldf.at[idx.at[j]], ld_v.at[dst], sem))
        for c in copies:
            c.wait()
        pltpu.sync_copy(z_v, out.at[pl.ds(m * BATCH + base, _BPW)])

        def body(r, _):
            for c in range(DIM // _L):
                s = pl.ds(c * _L, _L)
                ld_v[r, s] = z_v[r, s] + jnp.exp(ld_v[r, s])
            return 0

        lax.fori_loop(0, _BPW, body, 0)
        pltpu.sync_copy(ld_v, out.at[pl.ds((NUM_MODELS + m) * BATCH + base, _BPW)])


def kernel(z, logdelta, ids):
    zf = z.reshape(NUM_MODELS * NUM_BOXES, DIM)
    ldf = logdelta.reshape(NUM_MODELS * NUM_BOXES, DIM)
    ids3 = ids.astype(jnp.int32).reshape(_NW, _NCH, _CHUNK)
    mesh = plsc.VectorSubcoreMesh(core_axis_name="c", subcore_axis_name="s")
    out = pl.kernel(
        _sc_body,
        mesh=mesh,
        out_type=jax.ShapeDtypeStruct((2 * NUM_MODELS * BATCH, DIM), jnp.float32),
        scratch_types=[
            pltpu.VMEM((_NCH, _CHUNK), jnp.int32),
            pltpu.VMEM((_NCH, _CHUNK), jnp.int32),
            pltpu.VMEM((_BPW, DIM), jnp.float32),
            pltpu.VMEM((_BPW, DIM), jnp.float32),
            pltpu.SemaphoreType.DMA,
        ],
        compiler_params=pltpu.CompilerParams(use_tc_tiling_on_sc=False),
    )(zf, ldf, ids3)
    return out.reshape(2, NUM_MODELS, BATCH, DIM)


# speedup vs baseline: 1.0006x; 1.0006x over previous
import jax
import jax.numpy as jnp
from jax import lax
from jax.experimental import pallas as pl
from jax.experimental.pallas import tpu as pltpu
from jax.experimental.pallas import tpu_sc as plsc

NUM_MODELS = 2
NUM_BOXES = 100000
DIM = 64
BATCH = 16384

_L = 16
_NC, _NS = 2, 16
_NW = _NC * _NS
_BPW = BATCH // _NW
_CHUNK = 128
_NCH = _BPW // _CHUNK


def _sc_body(zf, ldf, ids, out, ids_v, ids1_v, z_v, ld_v, sem):
    wid = lax.axis_index("s") * _NC + lax.axis_index("c")
    base = wid * _BPW
    pltpu.sync_copy(ids.at[pl.ds(base, _BPW)], ids_v)
    for i in range(_BPW // _L):
        s = pl.ds(i * _L, _L)
        ids1_v[s] = ids_v[s] + NUM_BOXES
    for m, idx in ((0, ids_v), (1, ids1_v)):
        copies = []
        for j in range(_NCH):
            sl = pl.ds(j * _CHUNK, _CHUNK)
            copies.append(pltpu.async_copy(zf.at[idx.at[sl]], z_v.at[sl], sem))
            copies.append(pltpu.async_copy(ldf.at[idx.at[sl]], ld_v.at[sl], sem))
        for c in copies:
            c.wait()
        pltpu.sync_copy(z_v, out.at[0, m, pl.ds(base, _BPW)])

        def body(r, _):
            for c in range(DIM // _L):
                s = pl.ds(c * _L, _L)
                ld_v[r, s] = z_v[r, s] + jnp.exp(ld_v[r, s])
            return 0

        lax.fori_loop(0, _BPW, body, 0)
        pltpu.sync_copy(ld_v, out.at[1, m, pl.ds(base, _BPW)])


def kernel(z, logdelta, ids):
    zf = z.reshape(NUM_MODELS * NUM_BOXES, DIM)
    ldf = logdelta.reshape(NUM_MODELS * NUM_BOXES, DIM)
    ids32 = ids.astype(jnp.int32)
    mesh = plsc.VectorSubcoreMesh(core_axis_name="c", subcore_axis_name="s")
    out = pl.kernel(
        _sc_body,
        mesh=mesh,
        out_type=jax.ShapeDtypeStruct((2, NUM_MODELS, BATCH, DIM), jnp.float32),
        scratch_types=[
            pltpu.VMEM((_BPW,), jnp.int32),
            pltpu.VMEM((_BPW,), jnp.int32),
            pltpu.VMEM((_BPW, DIM), jnp.float32),
            pltpu.VMEM((_BPW, DIM), jnp.float32),
            pltpu.SemaphoreType.DMA,
        ],
        compiler_params=pltpu.CompilerParams(use_tc_tiling_on_sc=False),
    )(zf, ldf, ids32)
    return out


# speedup vs baseline: 1.0016x; 1.0010x over previous
import jax
import jax.numpy as jnp
from jax import lax
from jax.experimental import pallas as pl
from jax.experimental.pallas import tpu as pltpu
from jax.experimental.pallas import tpu_sc as plsc

NUM_MODELS = 2
NUM_BOXES = 100000
DIM = 64
BATCH = 16384

_L = 16
_NC, _NS = 2, 16
_NW = _NC * _NS
_BPW = BATCH // _NW
_CHUNK = 128
_NCH = _BPW // _CHUNK


def _sc_body(zf, ldf, ids, out, ids_v, z_v, ld_v, sem):
    wid = lax.axis_index("s") * _NC + lax.axis_index("c")
    base = wid * _BPW
    pltpu.sync_copy(ids.at[pl.ds(base, _BPW)], ids_v)
    for m in range(NUM_MODELS):
        zm, ldm = zf.at[m], ldf.at[m]
        copies = []
        for j in range(_NCH):
            sl = pl.ds(j * _CHUNK, _CHUNK)
            copies.append(pltpu.async_copy(zm.at[ids_v.at[sl]], z_v.at[sl], sem))
            copies.append(pltpu.async_copy(ldm.at[ids_v.at[sl]], ld_v.at[sl], sem))
        for c in copies:
            c.wait()
        pltpu.sync_copy(z_v, out.at[0, m, pl.ds(base, _BPW)])

        def body(r, _):
            for c in range(DIM // _L):
                s = pl.ds(c * _L, _L)
                ld_v[r, s] = z_v[r, s] + jnp.exp(ld_v[r, s])
            return 0

        lax.fori_loop(0, _BPW, body, 0)
        pltpu.sync_copy(ld_v, out.at[1, m, pl.ds(base, _BPW)])


def kernel(z, logdelta, ids):
    ids32 = ids.astype(jnp.int32)
    mesh = plsc.VectorSubcoreMesh(core_axis_name="c", subcore_axis_name="s")
    out = pl.kernel(
        _sc_body,
        mesh=mesh,
        out_type=jax.ShapeDtypeStruct((2, NUM_MODELS, BATCH, DIM), jnp.float32),
        scratch_types=[
            pltpu.VMEM((_BPW,), jnp.int32),
            pltpu.VMEM((_BPW, DIM), jnp.float32),
            pltpu.VMEM((_BPW, DIM), jnp.float32),
            pltpu.SemaphoreType.DMA,
        ],
        compiler_params=pltpu.CompilerParams(use_tc_tiling_on_sc=False),
    )(z, logdelta, ids32)
    return out


# speedup vs baseline: 1.3698x; 1.3677x over previous
import jax
import jax.numpy as jnp
from jax import lax
from jax.experimental import pallas as pl
from jax.experimental.pallas import tpu as pltpu
from jax.experimental.pallas import tpu_sc as plsc

NUM_MODELS = 2
NUM_BOXES = 100000
DIM = 64
BATCH = 16384

_L = 16
_NC, _NS = 2, 16
_NW = _NC * _NS
_BPW = BATCH // _NW
_CH = 32
_NCH = _BPW // _CH


def _sc_body(z, ld, ids, out, ids_v, zb, ldb, gsem, wsem):
    wid = lax.axis_index("s") * _NC + lax.axis_index("c")
    base = wid * _BPW
    pltpu.sync_copy(ids.at[pl.ds(base, _BPW)], ids_v)
    hsrc = z.at[0].at[pl.ds(0, _CH)]

    for ch in range(_NCH):
        p = ch % 2
        if ch >= 2:
            for m in range(NUM_MODELS):
                pltpu.make_async_copy(hsrc, zb.at[p, m], wsem).wait()
                pltpu.make_async_copy(hsrc, ldb.at[p, m], wsem).wait()

        def fire(g, _):
            v = ids_v[pl.ds(ch * _CH + g * _L, _L)]
            for j in range(_L):
                rid = v[j]
                row = pl.ds(rid, 1)
                dst = pl.ds(g * _L + j, 1)
                for m in range(NUM_MODELS):
                    pltpu.async_copy(z.at[m].at[row], zb.at[p, m, dst], gsem)
                    pltpu.async_copy(ld.at[m].at[row], ldb.at[p, m, dst], gsem)
            return 0

        lax.fori_loop(0, _CH // _L, fire, 0)
        for m in range(NUM_MODELS):
            pltpu.make_async_copy(hsrc, zb.at[p, m], gsem).wait()
            pltpu.make_async_copy(hsrc, ldb.at[p, m], gsem).wait()

        def body(r, _):
            for m in range(NUM_MODELS):
                for c in range(DIM // _L):
                    s = pl.ds(c * _L, _L)
                    ldb[p, m, r, s] = zb[p, m, r, s] + jnp.exp(ldb[p, m, r, s])
            return 0

        lax.fori_loop(0, _CH, body, 0)

        orow = pl.ds(base + ch * _CH, _CH)
        for m in range(NUM_MODELS):
            pltpu.async_copy(zb.at[p, m], out.at[0, m, orow], wsem)
            pltpu.async_copy(ldb.at[p, m], out.at[1, m, orow], wsem)

    for p in range(2):
        for m in range(NUM_MODELS):
            pltpu.make_async_copy(hsrc, zb.at[p, m], wsem).wait()
            pltpu.make_async_copy(hsrc, ldb.at[p, m], wsem).wait()


def kernel(z, logdelta, ids):
    ids32 = ids.astype(jnp.int32)
    mesh = plsc.VectorSubcoreMesh(core_axis_name="c", subcore_axis_name="s")
    out = pl.kernel(
        _sc_body,
        mesh=mesh,
        out_type=jax.ShapeDtypeStruct((2, NUM_MODELS, BATCH, DIM), jnp.float32),
        scratch_types=[
            pltpu.VMEM((_BPW,), jnp.int32),
            pltpu.VMEM((2, NUM_MODELS, _CH, DIM), jnp.float32),
            pltpu.VMEM((2, NUM_MODELS, _CH, DIM), jnp.float32),
            pltpu.SemaphoreType.DMA,
            pltpu.SemaphoreType.DMA,
        ],
    )(z, logdelta, ids32)
    return out


# speedup vs baseline: 1.4435x; 1.0538x over previous
import jax
import jax.numpy as jnp
from jax import lax
from jax.experimental import pallas as pl
from jax.experimental.pallas import tpu as pltpu
from jax.experimental.pallas import tpu_sc as plsc

NUM_MODELS = 2
NUM_BOXES = 100000
DIM = 64
BATCH = 16384

_L = 16
_NC, _NS = 2, 16
_NW = _NC * _NS
_BPW = BATCH // _NW
_CH = 32
_NCH = _BPW // _CH


def _sc_body(z, ld, ids, out, ids_v, zb, ldb, gsems, wsems):
    wid = lax.axis_index("s") * _NC + lax.axis_index("c")
    base = wid * _BPW
    pltpu.sync_copy(ids.at[pl.ds(base, _BPW)], ids_v)
    hsrc = z.at[0].at[pl.ds(0, _CH)]

    def fire(ch):
        p = ch % 2

        def go(g, _):
            v = ids_v[pl.ds(ch * _CH + g * _L, _L)]
            for j in range(_L):
                row = pl.ds(v[j], 1)
                dst = pl.ds(g * _L + j, 1)
                for m in range(NUM_MODELS):
                    pltpu.async_copy(z.at[m].at[row], zb.at[p, m, dst], gsems.at[p])
                    pltpu.async_copy(ld.at[m].at[row], ldb.at[p, m, dst], gsems.at[p])
            return 0

        lax.fori_loop(0, _CH // _L, go, 0)

    fire(0)
    for ch in range(_NCH):
        p = ch % 2
        if ch + 1 < _NCH:
            p2 = (ch + 1) % 2
            if ch + 1 >= 2:
                for m in range(NUM_MODELS):
                    pltpu.make_async_copy(hsrc, zb.at[p2, m], wsems.at[p2]).wait()
                    pltpu.make_async_copy(hsrc, ldb.at[p2, m], wsems.at[p2]).wait()
            fire(ch + 1)

        for m in range(NUM_MODELS):
            pltpu.make_async_copy(hsrc, zb.at[p, m], gsems.at[p]).wait()
            pltpu.make_async_copy(hsrc, ldb.at[p, m], gsems.at[p]).wait()

        def body(r, _):
            for m in range(NUM_MODELS):
                for c in range(DIM // _L):
                    s = pl.ds(c * _L, _L)
                    ldb[p, m, r, s] = zb[p, m, r, s] + jnp.exp(ldb[p, m, r, s])
            return 0

        lax.fori_loop(0, _CH, body, 0)

        orow = pl.ds(base + ch * _CH, _CH)
        for m in range(NUM_MODELS):
            pltpu.async_copy(zb.at[p, m], out.at[0, m, orow], wsems.at[p])
            pltpu.async_copy(ldb.at[p, m], out.at[1, m, orow], wsems.at[p])

    for p in range(2):
        for m in range(NUM_MODELS):
            pltpu.make_async_copy(hsrc, zb.at[p, m], wsems.at[p]).wait()
            pltpu.make_async_copy(hsrc, ldb.at[p, m], wsems.at[p]).wait()


def kernel(z, logdelta, ids):
    ids32 = ids.astype(jnp.int32)
    mesh = plsc.VectorSubcoreMesh(core_axis_name="c", subcore_axis_name="s")
    out = pl.kernel(
        _sc_body,
        mesh=mesh,
        out_type=jax.ShapeDtypeStruct((2, NUM_MODELS, BATCH, DIM), jnp.float32),
        scratch_types=[
            pltpu.VMEM((_BPW,), jnp.int32),
            pltpu.VMEM((2, NUM_MODELS, _CH, DIM), jnp.float32),
            pltpu.VMEM((2, NUM_MODELS, _CH, DIM), jnp.float32),
            pltpu.SemaphoreType.DMA((2,)),
            pltpu.SemaphoreType.DMA((2,)),
        ],
    )(z, logdelta, ids32)
    return out


# speedup vs baseline: 1.4562x; 1.0088x over previous
import jax
import jax.numpy as jnp
from jax import lax
from jax.experimental import pallas as pl
from jax.experimental.pallas import tpu as pltpu
from jax.experimental.pallas import tpu_sc as plsc

NUM_MODELS = 2
NUM_BOXES = 100000
DIM = 64
BATCH = 16384

_L = 16
_NC, _NS = 2, 16
_NW = _NC * _NS
_BPW = BATCH // _NW
_CH = 64
_NCH = _BPW // _CH


def _sc_body(z, ld, ids, out, ids_v, zb, ldb, gsems, wsems):
    wid = lax.axis_index("s") * _NC + lax.axis_index("c")
    base = wid * _BPW
    pltpu.sync_copy(ids.at[pl.ds(base, _BPW)], ids_v)
    hsrc = z.at[0].at[pl.ds(0, _CH)]

    def fire(ch):
        p = ch % 2

        def go(g, _):
            v = ids_v[pl.ds(ch * _CH + g * _L, _L)]
            for j in range(_L):
                row = pl.ds(v[j], 1)
                dst = pl.ds(g * _L + j, 1)
                for m in range(NUM_MODELS):
                    pltpu.async_copy(z.at[m].at[row], zb.at[p, m, dst], gsems.at[p])
                    pltpu.async_copy(ld.at[m].at[row], ldb.at[p, m, dst], gsems.at[p])
            return 0

        lax.fori_loop(0, _CH // _L, go, 0)

    fire(0)
    for ch in range(_NCH):
        p = ch % 2
        if ch + 1 < _NCH:
            p2 = (ch + 1) % 2
            if ch + 1 >= 2:
                for m in range(NUM_MODELS):
                    pltpu.make_async_copy(hsrc, zb.at[p2, m], wsems.at[p2]).wait()
                    pltpu.make_async_copy(hsrc, ldb.at[p2, m], wsems.at[p2]).wait()
            fire(ch + 1)

        for m in range(NUM_MODELS):
            pltpu.make_async_copy(hsrc, zb.at[p, m], gsems.at[p]).wait()
            pltpu.make_async_copy(hsrc, ldb.at[p, m], gsems.at[p]).wait()

        def body(r, _):
            for m in range(NUM_MODELS):
                for c in range(DIM // _L):
                    s = pl.ds(c * _L, _L)
                    ldb[p, m, r, s] = zb[p, m, r, s] + jnp.exp(ldb[p, m, r, s])
            return 0

        lax.fori_loop(0, _CH, body, 0)

        orow = pl.ds(base + ch * _CH, _CH)
        for m in range(NUM_MODELS):
            pltpu.async_copy(zb.at[p, m], out.at[0, m, orow], wsems.at[p])
            pltpu.async_copy(ldb.at[p, m], out.at[1, m, orow], wsems.at[p])

    for p in range(2):
        for m in range(NUM_MODELS):
            pltpu.make_async_copy(hsrc, zb.at[p, m], wsems.at[p]).wait()
            pltpu.make_async_copy(hsrc, ldb.at[p, m], wsems.at[p]).wait()


def kernel(z, logdelta, ids):
    ids32 = ids.astype(jnp.int32)
    mesh = plsc.VectorSubcoreMesh(core_axis_name="c", subcore_axis_name="s")
    out = pl.kernel(
        _sc_body,
        mesh=mesh,
        out_type=jax.ShapeDtypeStruct((2, NUM_MODELS, BATCH, DIM), jnp.float32),
        scratch_types=[
            pltpu.VMEM((_BPW,), jnp.int32),
            pltpu.VMEM((2, NUM_MODELS, _CH, DIM), jnp.float32),
            pltpu.VMEM((2, NUM_MODELS, _CH, DIM), jnp.float32),
            pltpu.SemaphoreType.DMA((2,)),
            pltpu.SemaphoreType.DMA((2,)),
        ],
    )(z, logdelta, ids32)
    return out
